# Initial kernel scaffold; baseline (speedup 1.0000x reference)
import jax
import jax.numpy as jnp
from jax.experimental import pallas as pl


def kernel(x, edge_index, W1, b1, W2, b2, Wf1, bf1, Wf2, bf2):
    raise NotImplementedError("write your pallas kernel here")


# speedup vs baseline: 169.2926x; 169.2926x over previous
import functools

import jax
import jax.numpy as jnp
from jax import lax
from jax.experimental import pallas as pl
from jax.experimental.pallas import tpu as pltpu
from jax.experimental.pallas import tpu_sc as plsc

N = 50000
E = 800000
H = 64
HALF = 32
FFN_D = 128
OUT_D = 32

N_PAD = 50176
E_PAD = 802816
NPT = N_PAD // 16
CHUNK = 128
ROWS16 = E_PAD // (16 * CHUNK)
ROWS32 = E_PAD // (32 * CHUNK)
STG = 224
IB = 8
NBLK = ROWS16 // IB
IB_DEG = 14
NBLK_DEG = ROWS32 // IB_DEG

_mesh = plsc.VectorSubcoreMesh(core_axis_name="c", subcore_axis_name="s")
_sc_params = pltpu.CompilerParams(use_tc_tiling_on_sc=False)


@functools.partial(
    pl.kernel,
    out_type=(
        jax.ShapeDtypeStruct((N_PAD,), jnp.float32),
        jax.ShapeDtypeStruct((N_PAD,), jnp.float32),
    ),
    mesh=_mesh,
    compiler_params=_sc_params,
    scratch_types=[
        pltpu.VMEM((IB_DEG, CHUNK), jnp.int32),
        pltpu.VMEM((CHUNK,), jnp.float32),
        pltpu.VMEM((NPT,), jnp.float32),
        pltpu.VMEM_SHARED((N_PAD,), jnp.float32),
        pltpu.SemaphoreType.DMA,
    ],
)
def _sc_degree(dst32_hbm, ones_hbm, zeros_hbm, deg0_hbm, deg1_hbm,
               dst_blk, ones_v, zbuf, deg_sh, sem):
    i32 = jnp.int32
    cid = lax.axis_index("c")
    sid = lax.axis_index("s")
    wid = cid * i32(16) + sid
    row0 = sid * i32(NPT)
    pltpu.sync_copy(ones_hbm, ones_v)
    pltpu.sync_copy(zeros_hbm.at[pl.ds(row0, NPT)], zbuf)
    pltpu.sync_copy(zbuf, deg_sh.at[pl.ds(row0, NPT)])
    plsc.subcore_barrier()

    def body(b, carry):
        pltpu.sync_copy(dst32_hbm.at[wid, pl.ds(b * i32(IB_DEG), IB_DEG)], dst_blk)
        for k in range(IB_DEG):
            pltpu.async_copy(ones_v, deg_sh.at[dst_blk.at[i32(k)]], sem, add=True)
        for k in range(IB_DEG):
            pltpu.make_async_copy(ones_hbm, ones_v, sem).wait()
        return carry

    lax.fori_loop(i32(0), i32(NBLK_DEG), body, i32(0))
    plsc.subcore_barrier()

    pltpu.sync_copy(deg_sh.at[pl.ds(row0, NPT)], zbuf)

    @pl.when(cid == 0)
    def _():
        pltpu.sync_copy(zbuf, deg0_hbm.at[pl.ds(row0, NPT)])

    @pl.when(cid == 1)
    def _():
        pltpu.sync_copy(zbuf, deg1_hbm.at[pl.ds(row0, NPT)])


@functools.partial(
    pl.kernel,
    out_type=(
        jax.ShapeDtypeStruct((N_PAD, HALF), jnp.float32),
        jax.ShapeDtypeStruct((N_PAD, HALF), jnp.float32),
    ),
    mesh=_mesh,
    compiler_params=_sc_params,
    scratch_types=[
        pltpu.VMEM((IB, CHUNK), jnp.int32),
        pltpu.VMEM((IB, CHUNK), jnp.int32),
        pltpu.VMEM((CHUNK, HALF), jnp.float32),
        pltpu.VMEM((CHUNK, HALF), jnp.float32),
        pltpu.VMEM((STG, HALF), jnp.float32),
        pltpu.VMEM_SHARED((N_PAD, HALF), jnp.float32),
        pltpu.SemaphoreType.DMA,
        pltpu.SemaphoreType.DMA,
    ],
)
def _sc_aggregate(y0_hbm, y1_hbm, src16_hbm, dst16_hbm, o0_hbm, o1_hbm,
                  src_blk, dst_blk, rows0, rows1, stg_v, acc_sh, sem0, sem1):
    i32 = jnp.int32
    cid = lax.axis_index("c")
    sid = lax.axis_index("s")
    row0 = sid * i32(NPT)

    def stage_in(y_hbm):
        def body(t, carry):
            r = row0 + t * i32(STG)
            pltpu.sync_copy(y_hbm.at[pl.ds(r, STG)], stg_v)
            pltpu.sync_copy(stg_v, acc_sh.at[pl.ds(r, STG)])
            return carry

        lax.fori_loop(i32(0), i32(NPT // STG), body, i32(0))

    @pl.when(cid == 0)
    def _():
        stage_in(y0_hbm)

    @pl.when(cid == 1)
    def _():
        stage_in(y1_hbm)

    plsc.subcore_barrier()

    def edge_loop(y_hbm):
        def body(b, carry):
            r = b * i32(IB)
            pltpu.sync_copy(src16_hbm.at[sid, pl.ds(r, IB)], src_blk)
            pltpu.sync_copy(dst16_hbm.at[sid, pl.ds(r, IB)], dst_blk)
            for g in range(IB // 2):
                j0 = i32(2 * g)
                j1 = i32(2 * g + 1)
                d0 = pltpu.async_copy(y_hbm.at[src_blk.at[j0]], rows0, sem0)
                d1 = pltpu.async_copy(y_hbm.at[src_blk.at[j1]], rows1, sem1)
                d0.wait()
                pltpu.sync_copy(rows0, acc_sh.at[dst_blk.at[j0]], add=True)
                d1.wait()
                pltpu.sync_copy(rows1, acc_sh.at[dst_blk.at[j1]], add=True)
            return carry

        lax.fori_loop(i32(0), i32(NBLK), body, i32(0))

    @pl.when(cid == 0)
    def _():
        edge_loop(y0_hbm)

    @pl.when(cid == 1)
    def _():
        edge_loop(y1_hbm)

    plsc.subcore_barrier()

    def stage_out(o_hbm):
        def body(t, carry):
            r = row0 + t * i32(STG)
            pltpu.sync_copy(acc_sh.at[pl.ds(r, STG)], stg_v)
            pltpu.sync_copy(stg_v, o_hbm.at[pl.ds(r, STG)])
            return carry

        lax.fori_loop(i32(0), i32(NPT // STG), body, i32(0))

    @pl.when(cid == 0)
    def _():
        stage_out(o0_hbm)

    @pl.when(cid == 1)
    def _():
        stage_out(o1_hbm)


R_TC = 1568
GRID = N_PAD // R_TC


def _rows_spec(width):
    return pl.BlockSpec((R_TC, width), lambda i: (i, jnp.int32(0)))


def _full_spec(shape):
    return pl.BlockSpec(shape, lambda i: tuple(jnp.int32(0) for _ in shape))


def _tc_prologue_body(dp_ref, x_ref, w1_ref, dinv_ref, y0_ref, y1_ref):
    dp = dp_ref[...]
    deg = dp[:, 0:1] + dp[:, 1:2] + 1.0
    dinv = lax.rsqrt(deg)
    x = x_ref[...]
    w = w1_ref[...]
    xw = x[:, 0:1] * w[0:1, :] + x[:, 1:2] * w[1:2, :]
    y = xw * dinv
    dinv_ref[...] = dinv
    y0_ref[...] = y[:, :HALF]
    y1_ref[...] = y[:, HALF:]


def _tc_mid_body(a0_ref, a1_ref, dinv_ref, b_ref, w_ref, o0_ref, o1_ref):
    dinv = dinv_ref[...]
    h = jnp.concatenate([a0_ref[...], a1_ref[...]], axis=1) * dinv + b_ref[...]
    h = jnp.maximum(h, 0.0)
    xw = jnp.dot(h, w_ref[...], preferred_element_type=jnp.float32, precision=lax.Precision.HIGHEST)
    y = xw * dinv
    o0_ref[...] = y[:, :HALF]
    o1_ref[...] = y[:, HALF:]


def _tc_final_body(c0_ref, c1_ref, dinv_ref, b2_ref, wf1_ref, bf1_ref,
                   wf2_ref, bf2_ref, o_ref):
    dinv = dinv_ref[...]
    h = jnp.concatenate([c0_ref[...], c1_ref[...]], axis=1) * dinv + b2_ref[...]
    h = jnp.maximum(h, 0.0)
    f = jnp.dot(h, wf1_ref[...], preferred_element_type=jnp.float32, precision=lax.Precision.HIGHEST) + bf1_ref[...]
    f = jnp.maximum(f, 0.0)
    o_ref[...] = jnp.dot(f, wf2_ref[...], preferred_element_type=jnp.float32, precision=lax.Precision.HIGHEST) + bf2_ref[...]


def _tc_prologue(deg_nodes, x_pad, W1):
    return pl.pallas_call(
        _tc_prologue_body,
        grid=(GRID,),
        in_specs=[_rows_spec(2), _rows_spec(2), _full_spec((2, H))],
        out_specs=(_rows_spec(1), _rows_spec(HALF), _rows_spec(HALF)),
        out_shape=(
            jax.ShapeDtypeStruct((N_PAD, 1), jnp.float32),
            jax.ShapeDtypeStruct((N_PAD, HALF), jnp.float32),
            jax.ShapeDtypeStruct((N_PAD, HALF), jnp.float32),
        ),
    )(deg_nodes, x_pad, W1)


def _tc_mid(a0, a1, dinv, b1, W2):
    return pl.pallas_call(
        _tc_mid_body,
        grid=(GRID,),
        in_specs=[_rows_spec(HALF), _rows_spec(HALF), _rows_spec(1),
                  _full_spec((1, H)), _full_spec((H, H))],
        out_specs=(_rows_spec(HALF), _rows_spec(HALF)),
        out_shape=(
            jax.ShapeDtypeStruct((N_PAD, HALF), jnp.float32),
            jax.ShapeDtypeStruct((N_PAD, HALF), jnp.float32),
        ),
    )(a0, a1, dinv, b1, W2)


def _tc_final(c0, c1, dinv, b2, Wf1, bf1, Wf2, bf2):
    return pl.pallas_call(
        _tc_final_body,
        grid=(GRID,),
        in_specs=[_rows_spec(HALF), _rows_spec(HALF), _rows_spec(1),
                  _full_spec((1, H)), _full_spec((H, FFN_D)),
                  _full_spec((1, FFN_D)), _full_spec((FFN_D, OUT_D)),
                  _full_spec((1, OUT_D))],
        out_specs=_rows_spec(OUT_D),
        out_shape=jax.ShapeDtypeStruct((N_PAD, OUT_D), jnp.float32),
    )(c0, c1, dinv, b2, Wf1, bf1, Wf2, bf2)


def kernel(x, edge_index, W1, b1, W2, b2, Wf1, bf1, Wf2, bf2):
    f32 = jnp.float32
    x = x.astype(f32)
    src = edge_index[0].astype(jnp.int32)
    dst = edge_index[1].astype(jnp.int32)

    pad = jnp.int32(N) + (jnp.arange(E_PAD - E, dtype=jnp.int32) % (N_PAD - N))
    src_p = jnp.concatenate([src, pad])
    dst_p = jnp.concatenate([dst, pad])
    src16 = src_p.reshape(16, ROWS16, CHUNK)
    dst16 = dst_p.reshape(16, ROWS16, CHUNK)
    dst32 = dst_p.reshape(32, ROWS32, CHUNK)

    x_pad = jnp.zeros((N_PAD, 2), f32).at[:N].set(x)
    ones = jnp.ones((CHUNK,), f32)
    zeros = jnp.zeros((N_PAD,), f32)

    deg0, deg1 = _sc_degree(dst32, ones, zeros)
    deg_nodes = jnp.stack([deg0, deg1], axis=1)

    dinv, y0, y1 = _tc_prologue(deg_nodes, x_pad, W1.astype(f32))
    a0, a1 = _sc_aggregate(y0, y1, src16, dst16)
    z0, z1 = _tc_mid(a0, a1, dinv, b1.reshape(1, H).astype(f32), W2.astype(f32))
    c0, c1 = _sc_aggregate(z0, z1, src16, dst16)
    out = _tc_final(c0, c1, dinv, b2.reshape(1, H).astype(f32),
                    Wf1.astype(f32), bf1.reshape(1, FFN_D).astype(f32),
                    Wf2.astype(f32), bf2.reshape(1, OUT_D).astype(f32))
    return out[:N].astype(jnp.float64)


# speedup vs baseline: 194.3016x; 1.1477x over previous
import functools

import jax
import jax.numpy as jnp
from jax import lax
from jax.experimental import pallas as pl
from jax.experimental.pallas import tpu as pltpu
from jax.experimental.pallas import tpu_sc as plsc

N = 50000
E = 800000
H = 64
HALF = 32
FFN_D = 128
OUT_D = 32

N_PAD = 50176
E_PAD = 802816
NPT = N_PAD // 16
CHUNK = 128
ROWS16 = E_PAD // (16 * CHUNK)
ROWS32 = E_PAD // (32 * CHUNK)
STG = 224
IB = 8
NBLK = ROWS16 // IB
IB_DEG = 14
NBLK_DEG = ROWS32 // IB_DEG

N_P4 = N_PAD // 4

_mesh = plsc.VectorSubcoreMesh(core_axis_name="c", subcore_axis_name="s")
_sc_params = pltpu.CompilerParams(use_tc_tiling_on_sc=False)


@functools.partial(
    pl.kernel,
    out_type=(
        jax.ShapeDtypeStruct((N_PAD, HALF), jnp.float32),
        jax.ShapeDtypeStruct((N_PAD, HALF), jnp.float32),
    ),
    mesh=_mesh,
    compiler_params=_sc_params,
    scratch_types=[
        pltpu.VMEM((IB_DEG, CHUNK), jnp.int32),
        pltpu.VMEM((CHUNK, HALF), jnp.float32),
        pltpu.VMEM((STG, HALF), jnp.float32),
        pltpu.VMEM_SHARED((N_PAD, HALF), jnp.float32),
        pltpu.SemaphoreType.DMA,
    ],
)
def _sc_degree(dst32_hbm, ones_hbm, zeros_hbm, deg0_hbm, deg1_hbm,
               dst_blk, ones_v, stg_v, deg_sh, sem):
    i32 = jnp.int32
    cid = lax.axis_index("c")
    sid = lax.axis_index("s")
    wid = cid * i32(16) + sid
    row0 = sid * i32(NPT)
    pltpu.sync_copy(ones_hbm, ones_v)
    pltpu.sync_copy(zeros_hbm, stg_v)

    def zbody(t, carry):
        pltpu.sync_copy(stg_v, deg_sh.at[pl.ds(row0 + t * i32(STG), STG)])
        return carry

    lax.fori_loop(i32(0), i32(NPT // STG), zbody, i32(0))
    plsc.subcore_barrier()

    def body(b, carry):
        pltpu.sync_copy(dst32_hbm.at[wid, pl.ds(b * i32(IB_DEG), IB_DEG)], dst_blk)
        for k in range(IB_DEG):
            pltpu.async_copy(ones_v, deg_sh.at[dst_blk.at[i32(k)]], sem, add=True)
        for k in range(IB_DEG):
            pltpu.make_async_copy(ones_hbm, ones_v, sem).wait()
        return carry

    lax.fori_loop(i32(0), i32(NBLK_DEG), body, i32(0))
    plsc.subcore_barrier()

    def stage_out(o_hbm):
        def obody(t, carry):
            r = row0 + t * i32(STG)
            pltpu.sync_copy(deg_sh.at[pl.ds(r, STG)], stg_v)
            pltpu.sync_copy(stg_v, o_hbm.at[pl.ds(r, STG)])
            return carry

        lax.fori_loop(i32(0), i32(NPT // STG), obody, i32(0))

    @pl.when(cid == 0)
    def _():
        stage_out(deg0_hbm)

    @pl.when(cid == 1)
    def _():
        stage_out(deg1_hbm)


@functools.partial(
    pl.kernel,
    out_type=(
        jax.ShapeDtypeStruct((N_PAD, HALF), jnp.float32),
        jax.ShapeDtypeStruct((N_PAD, HALF), jnp.float32),
    ),
    mesh=_mesh,
    compiler_params=_sc_params,
    scratch_types=[
        pltpu.VMEM((IB, CHUNK), jnp.int32),
        pltpu.VMEM((IB, CHUNK), jnp.int32),
        pltpu.VMEM((CHUNK, HALF), jnp.float32),
        pltpu.VMEM((CHUNK, HALF), jnp.float32),
        pltpu.VMEM((STG, HALF), jnp.float32),
        pltpu.VMEM_SHARED((N_PAD, HALF), jnp.float32),
        pltpu.SemaphoreType.DMA,
        pltpu.SemaphoreType.DMA,
    ],
)
def _sc_aggregate(y0_hbm, y1_hbm, src16_hbm, dst16_hbm, o0_hbm, o1_hbm,
                  src_blk, dst_blk, rows0, rows1, stg_v, acc_sh, sem0, sem1):
    i32 = jnp.int32
    cid = lax.axis_index("c")
    sid = lax.axis_index("s")
    row0 = sid * i32(NPT)

    def stage_in(y_hbm):
        def body(t, carry):
            r = row0 + t * i32(STG)
            pltpu.sync_copy(y_hbm.at[pl.ds(r, STG)], stg_v)
            pltpu.sync_copy(stg_v, acc_sh.at[pl.ds(r, STG)])
            return carry

        lax.fori_loop(i32(0), i32(NPT // STG), body, i32(0))

    @pl.when(cid == 0)
    def _():
        stage_in(y0_hbm)

    @pl.when(cid == 1)
    def _():
        stage_in(y1_hbm)

    plsc.subcore_barrier()

    def edge_loop(y_hbm):
        def body(b, carry):
            r = b * i32(IB)
            pltpu.sync_copy(src16_hbm.at[sid, pl.ds(r, IB)], src_blk)
            pltpu.sync_copy(dst16_hbm.at[sid, pl.ds(r, IB)], dst_blk)
            for g in range(IB // 2):
                j0 = i32(2 * g)
                j1 = i32(2 * g + 1)
                d0 = pltpu.async_copy(y_hbm.at[src_blk.at[j0]], rows0, sem0)
                d1 = pltpu.async_copy(y_hbm.at[src_blk.at[j1]], rows1, sem1)
                d0.wait()
                pltpu.sync_copy(rows0, acc_sh.at[dst_blk.at[j0]], add=True)
                d1.wait()
                pltpu.sync_copy(rows1, acc_sh.at[dst_blk.at[j1]], add=True)
            return carry

        lax.fori_loop(i32(0), i32(NBLK), body, i32(0))

    @pl.when(cid == 0)
    def _():
        edge_loop(y0_hbm)

    @pl.when(cid == 1)
    def _():
        edge_loop(y1_hbm)

    plsc.subcore_barrier()

    def stage_out(o_hbm):
        def body(t, carry):
            r = row0 + t * i32(STG)
            pltpu.sync_copy(acc_sh.at[pl.ds(r, STG)], stg_v)
            pltpu.sync_copy(stg_v, o_hbm.at[pl.ds(r, STG)])
            return carry

        lax.fori_loop(i32(0), i32(NPT // STG), body, i32(0))

    @pl.when(cid == 0)
    def _():
        stage_out(o0_hbm)

    @pl.when(cid == 1)
    def _():
        stage_out(o1_hbm)


R_P = 1568
GRID_P = N_P4 // R_P


def _prow_spec():
    return pl.BlockSpec((R_P, 128), lambda i: (i, jnp.int32(0)))


def _full_spec(shape):
    return pl.BlockSpec(shape, lambda i: tuple(jnp.int32(0) for _ in shape))


def _dot(a, b):
    return jnp.dot(a, b, preferred_element_type=jnp.float32,
                   precision=lax.Precision.HIGHEST)


def _tc_prologue_body(d0_ref, d1_ref, x0_ref, x1_ref, t_ref,
                      dinv_ref, y0_ref, y1_ref):
    deg = d0_ref[...] + d1_ref[...] + 1.0
    dinv = lax.rsqrt(deg)
    t = t_ref[...]
    x0 = x0_ref[...]
    x1 = x1_ref[...]
    y0 = (x0 * t[0:1, :] + x1 * t[1:2, :]) * dinv
    y1 = (x0 * t[2:3, :] + x1 * t[3:4, :]) * dinv
    dinv_ref[...] = dinv
    y0_ref[...] = y0
    y1_ref[...] = y1


def _tc_mid_body(a0_ref, a1_ref, dinv_ref, b_ref,
                 w00_ref, w01_ref, w10_ref, w11_ref, o0_ref, o1_ref):
    dinv = dinv_ref[...]
    b = b_ref[...]
    h0 = jnp.maximum(a0_ref[...] * dinv + b[0:1, :], 0.0)
    h1 = jnp.maximum(a1_ref[...] * dinv + b[1:2, :], 0.0)
    o0_ref[...] = (_dot(h0, w00_ref[...]) + _dot(h1, w10_ref[...])) * dinv
    o1_ref[...] = (_dot(h0, w01_ref[...]) + _dot(h1, w11_ref[...])) * dinv


def _tc_final_body(c0_ref, c1_ref, dinv_ref, b_ref,
                   wf1a_ref, wf1b_ref, bf1_ref, wf2_ref, bf2_ref, o_ref):
    dinv = dinv_ref[...]
    b = b_ref[...]
    h0 = jnp.maximum(c0_ref[...] * dinv + b[0:1, :], 0.0)
    h1 = jnp.maximum(c1_ref[...] * dinv + b[1:2, :], 0.0)
    f = _dot(h0, wf1a_ref[...]) + _dot(h1, wf1b_ref[...]) + bf1_ref[...]
    f = jnp.maximum(f, 0.0)
    o_ref[...] = _dot(f, wf2_ref[...]) + bf2_ref[...]


def _tc_prologue(d0, d1, x0, x1, t_w1):
    return pl.pallas_call(
        _tc_prologue_body,
        grid=(GRID_P,),
        in_specs=[_prow_spec(), _prow_spec(), _prow_spec(), _prow_spec(),
                  _full_spec((4, 128))],
        out_specs=(_prow_spec(), _prow_spec(), _prow_spec()),
        out_shape=(
            jax.ShapeDtypeStruct((N_P4, 128), jnp.float32),
            jax.ShapeDtypeStruct((N_P4, 128), jnp.float32),
            jax.ShapeDtypeStruct((N_P4, 128), jnp.float32),
        ),
    )(d0, d1, x0, x1, t_w1)


def _tc_mid(a0, a1, dinv, b1t, w00, w01, w10, w11):
    return pl.pallas_call(
        _tc_mid_body,
        grid=(GRID_P,),
        in_specs=[_prow_spec(), _prow_spec(), _prow_spec(),
                  _full_spec((2, 128)), _full_spec((128, 128)),
                  _full_spec((128, 128)), _full_spec((128, 128)),
                  _full_spec((128, 128))],
        out_specs=(_prow_spec(), _prow_spec()),
        out_shape=(
            jax.ShapeDtypeStruct((N_P4, 128), jnp.float32),
            jax.ShapeDtypeStruct((N_P4, 128), jnp.float32),
        ),
    )(a0, a1, dinv, b1t, w00, w01, w10, w11)


def _tc_final(c0, c1, dinv, b2t, wf1a, wf1b, bf1t, wf2bd, bf2t):
    return pl.pallas_call(
        _tc_final_body,
        grid=(GRID_P,),
        in_specs=[_prow_spec(), _prow_spec(), _prow_spec(),
                  _full_spec((2, 128)),
                  _full_spec((128, 4 * FFN_D)), _full_spec((128, 4 * FFN_D)),
                  _full_spec((1, 4 * FFN_D)),
                  _full_spec((4 * FFN_D, 128)), _full_spec((1, 128))],
        out_specs=_prow_spec(),
        out_shape=jax.ShapeDtypeStruct((N_P4, 128), jnp.float32),
    )(c0, c1, dinv, b2t, wf1a, wf1b, bf1t, wf2bd, bf2t)


def kernel(x, edge_index, W1, b1, W2, b2, Wf1, bf1, Wf2, bf2):
    f32 = jnp.float32
    x = x.astype(f32)
    src = edge_index[0].astype(jnp.int32)
    dst = edge_index[1].astype(jnp.int32)

    pad = jnp.int32(N) + (jnp.arange(E_PAD - E, dtype=jnp.int32) % (N_PAD - N))
    src_p = jnp.concatenate([src, pad])
    dst_p = jnp.concatenate([dst, pad])
    src16 = src_p.reshape(16, ROWS16, CHUNK)
    dst16 = dst_p.reshape(16, ROWS16, CHUNK)
    dst32 = dst_p.reshape(32, ROWS32, CHUNK)

    ones32 = jnp.ones((CHUNK, HALF), f32)
    zeros32 = jnp.zeros((STG, HALF), f32)

    xcol = jnp.concatenate([x, jnp.zeros((N_PAD - N, 2), f32)])
    xp0 = jnp.broadcast_to(xcol[:, 0:1], (N_PAD, HALF)).reshape(N_P4, 128)
    xp1 = jnp.broadcast_to(xcol[:, 1:2], (N_PAD, HALF)).reshape(N_P4, 128)

    W1f, W2f = W1.astype(f32), W2.astype(f32)
    Wf1f, Wf2f = Wf1.astype(f32), Wf2.astype(f32)
    b1f, b2f = b1.astype(f32), b2.astype(f32)
    bf1f, bf2f = bf1.astype(f32), bf2.astype(f32)
    eye4 = jnp.eye(4, dtype=f32)

    def bd4(a):
        return jnp.kron(eye4, a)

    t_w1 = jnp.stack([
        jnp.tile(W1f[0, :HALF], 4), jnp.tile(W1f[1, :HALF], 4),
        jnp.tile(W1f[0, HALF:], 4), jnp.tile(W1f[1, HALF:], 4)])
    w00 = bd4(W2f[:HALF, :HALF])
    w01 = bd4(W2f[:HALF, HALF:])
    w10 = bd4(W2f[HALF:, :HALF])
    w11 = bd4(W2f[HALF:, HALF:])
    b1t = jnp.stack([jnp.tile(b1f[:HALF], 4), jnp.tile(b1f[HALF:], 4)])
    b2t = jnp.stack([jnp.tile(b2f[:HALF], 4), jnp.tile(b2f[HALF:], 4)])
    wf1a = bd4(Wf1f[:HALF, :])
    wf1b = bd4(Wf1f[HALF:, :])
    bf1t = jnp.tile(bf1f, 4).reshape(1, 4 * FFN_D)
    wf2bd = bd4(Wf2f)
    bf2t = jnp.tile(bf2f, 4).reshape(1, 128)

    deg0, deg1 = _sc_degree(dst32, ones32, zeros32)
    dinv, y0p, y1p = _tc_prologue(deg0.reshape(N_P4, 128),
                                  deg1.reshape(N_P4, 128), xp0, xp1, t_w1)

    a0, a1 = _sc_aggregate(y0p.reshape(N_PAD, HALF), y1p.reshape(N_PAD, HALF),
                           src16, dst16)
    z0p, z1p = _tc_mid(a0.reshape(N_P4, 128), a1.reshape(N_P4, 128),
                       dinv, b1t, w00, w01, w10, w11)
    c0, c1 = _sc_aggregate(z0p.reshape(N_PAD, HALF), z1p.reshape(N_PAD, HALF),
                           src16, dst16)
    outp = _tc_final(c0.reshape(N_P4, 128), c1.reshape(N_P4, 128),
                     dinv, b2t, wf1a, wf1b, bf1t, wf2bd, bf2t)
    return outp.reshape(N_PAD, OUT_D)[:N].astype(jnp.float64)


# speedup vs baseline: 209.0849x; 1.0761x over previous
import functools

import jax
import jax.numpy as jnp
from jax import lax
from jax.experimental import pallas as pl
from jax.experimental.pallas import tpu as pltpu
from jax.experimental.pallas import tpu_sc as plsc

N = 50000
E = 800000
H = 64
HALF = 32
FFN_D = 128
OUT_D = 32

N_PAD = 50176
E_PAD = 819200
NPT = N_PAD // 16
CHUNK = 128
ROWS16 = E_PAD // (16 * CHUNK)
ROWS32 = E_PAD // (32 * CHUNK)
STG = 56
IB = 16
NBLK = ROWS16 // IB
NBUF = 4
IB_DEG = 10
NBLK_DEG = ROWS32 // IB_DEG

N_P4 = N_PAD // 4

_mesh = plsc.VectorSubcoreMesh(core_axis_name="c", subcore_axis_name="s")
_sc_params = pltpu.CompilerParams(use_tc_tiling_on_sc=False)


@functools.partial(
    pl.kernel,
    out_type=(
        jax.ShapeDtypeStruct((N_PAD, HALF), jnp.float32),
        jax.ShapeDtypeStruct((N_PAD, HALF), jnp.float32),
    ),
    mesh=_mesh,
    compiler_params=_sc_params,
    scratch_types=[
        pltpu.VMEM((IB_DEG, CHUNK), jnp.int32),
        pltpu.VMEM((CHUNK, HALF), jnp.float32),
        pltpu.VMEM((STG, HALF), jnp.float32),
        pltpu.VMEM_SHARED((N_PAD, HALF), jnp.float32),
        pltpu.SemaphoreType.DMA,
    ],
)
def _sc_degree(dst32_hbm, ones_hbm, zeros_hbm, deg0_hbm, deg1_hbm,
               dst_blk, ones_v, stg_v, deg_sh, sem):
    i32 = jnp.int32
    cid = lax.axis_index("c")
    sid = lax.axis_index("s")
    wid = cid * i32(16) + sid
    row0 = sid * i32(NPT)
    pltpu.sync_copy(ones_hbm, ones_v)
    pltpu.sync_copy(zeros_hbm, stg_v)

    def zbody(t, carry):
        pltpu.sync_copy(stg_v, deg_sh.at[pl.ds(row0 + t * i32(STG), STG)])
        return carry

    lax.fori_loop(i32(0), i32(NPT // STG), zbody, i32(0))
    plsc.subcore_barrier()

    def body(b, carry):
        pltpu.sync_copy(dst32_hbm.at[wid, pl.ds(b * i32(IB_DEG), IB_DEG)], dst_blk)
        for k in range(IB_DEG):
            pltpu.async_copy(ones_v, deg_sh.at[dst_blk.at[i32(k)]], sem, add=True)
        for k in range(IB_DEG):
            pltpu.make_async_copy(ones_hbm, ones_v, sem).wait()
        return carry

    lax.fori_loop(i32(0), i32(NBLK_DEG), body, i32(0))
    plsc.subcore_barrier()

    def stage_out(o_hbm):
        def obody(t, carry):
            r = row0 + t * i32(STG)
            pltpu.sync_copy(deg_sh.at[pl.ds(r, STG)], stg_v)
            pltpu.sync_copy(stg_v, o_hbm.at[pl.ds(r, STG)])
            return carry

        lax.fori_loop(i32(0), i32(NPT // STG), obody, i32(0))

    @pl.when(cid == 0)
    def _():
        stage_out(deg0_hbm)

    @pl.when(cid == 1)
    def _():
        stage_out(deg1_hbm)


@functools.partial(
    pl.kernel,
    out_type=(
        jax.ShapeDtypeStruct((N_PAD, HALF), jnp.float32),
        jax.ShapeDtypeStruct((N_PAD, HALF), jnp.float32),
    ),
    mesh=_mesh,
    compiler_params=_sc_params,
    scratch_types=[
        pltpu.VMEM((IB, CHUNK), jnp.int32),
        pltpu.VMEM((IB, CHUNK), jnp.int32),
        [pltpu.VMEM((CHUNK, HALF), jnp.float32) for _ in range(NBUF)],
        pltpu.VMEM((STG, HALF), jnp.float32),
        pltpu.VMEM_SHARED((N_PAD, HALF), jnp.float32),
        [pltpu.SemaphoreType.DMA for _ in range(NBUF)],
        [pltpu.SemaphoreType.DMA for _ in range(NBUF)],
    ],
)
def _sc_aggregate(y0_hbm, y1_hbm, src16_hbm, dst16_hbm, o0_hbm, o1_hbm,
                  src_blk, dst_blk, rows, stg_v, acc_sh, sems_g, sems_s):
    i32 = jnp.int32
    cid = lax.axis_index("c")
    sid = lax.axis_index("s")
    row0 = sid * i32(NPT)

    def stage_in(y_hbm):
        def body(t, carry):
            r = row0 + t * i32(STG)
            pltpu.sync_copy(y_hbm.at[pl.ds(r, STG)], stg_v)
            pltpu.sync_copy(stg_v, acc_sh.at[pl.ds(r, STG)])
            return carry

        lax.fori_loop(i32(0), i32(NPT // STG), body, i32(0))

    @pl.when(cid == 0)
    def _():
        stage_in(y0_hbm)

    @pl.when(cid == 1)
    def _():
        stage_in(y1_hbm)

    plsc.subcore_barrier()

    def edge_loop(y_hbm):
        def wait_scatter(k):
            pltpu.make_async_copy(
                rows[k], acc_sh.at[dst_blk.at[i32(0)]], sems_s[k]).wait()

        def body(b, carry):
            r = b * i32(IB)
            pltpu.sync_copy(src16_hbm.at[sid, pl.ds(r, IB)], src_blk)
            pltpu.sync_copy(dst16_hbm.at[sid, pl.ds(r, IB)], dst_blk)
            for gi in range(IB // NBUF):
                if gi > 0:
                    for k in range(NBUF):
                        wait_scatter(k)
                for k in range(NBUF):
                    pltpu.async_copy(
                        y_hbm.at[src_blk.at[i32(gi * NBUF + k)]],
                        rows[k], sems_g[k])
                for k in range(NBUF):
                    pltpu.make_async_copy(
                        y_hbm.at[src_blk.at[i32(0)]], rows[k], sems_g[k]).wait()
                    pltpu.async_copy(
                        rows[k], acc_sh.at[dst_blk.at[i32(gi * NBUF + k)]],
                        sems_s[k], add=True)
            for k in range(NBUF):
                wait_scatter(k)
            return carry

        lax.fori_loop(i32(0), i32(NBLK), body, i32(0))

    @pl.when(cid == 0)
    def _():
        edge_loop(y0_hbm)

    @pl.when(cid == 1)
    def _():
        edge_loop(y1_hbm)

    plsc.subcore_barrier()

    def stage_out(o_hbm):
        def body(t, carry):
            r = row0 + t * i32(STG)
            pltpu.sync_copy(acc_sh.at[pl.ds(r, STG)], stg_v)
            pltpu.sync_copy(stg_v, o_hbm.at[pl.ds(r, STG)])
            return carry

        lax.fori_loop(i32(0), i32(NPT // STG), body, i32(0))

    @pl.when(cid == 0)
    def _():
        stage_out(o0_hbm)

    @pl.when(cid == 1)
    def _():
        stage_out(o1_hbm)


R_P = 1568
GRID_P = N_P4 // R_P


def _prow_spec():
    return pl.BlockSpec((R_P, 128), lambda i: (i, jnp.int32(0)))


def _full_spec(shape):
    return pl.BlockSpec(shape, lambda i: tuple(jnp.int32(0) for _ in shape))


def _dot(a, b):
    return jnp.dot(a, b, preferred_element_type=jnp.float32,
                   precision=lax.Precision.HIGHEST)


def _tc_prologue_body(d0_ref, d1_ref, x0_ref, x1_ref, t_ref,
                      dinv_ref, y0_ref, y1_ref):
    deg = d0_ref[...] + d1_ref[...] + 1.0
    dinv = lax.rsqrt(deg)
    t = t_ref[...]
    x0 = x0_ref[...]
    x1 = x1_ref[...]
    y0 = (x0 * t[0:1, :] + x1 * t[1:2, :]) * dinv
    y1 = (x0 * t[2:3, :] + x1 * t[3:4, :]) * dinv
    dinv_ref[...] = dinv
    y0_ref[...] = y0
    y1_ref[...] = y1


def _tc_mid_body(a0_ref, a1_ref, dinv_ref, b_ref,
                 w00_ref, w01_ref, w10_ref, w11_ref, o0_ref, o1_ref):
    dinv = dinv_ref[...]
    b = b_ref[...]
    h0 = jnp.maximum(a0_ref[...] * dinv + b[0:1, :], 0.0)
    h1 = jnp.maximum(a1_ref[...] * dinv + b[1:2, :], 0.0)
    o0_ref[...] = (_dot(h0, w00_ref[...]) + _dot(h1, w10_ref[...])) * dinv
    o1_ref[...] = (_dot(h0, w01_ref[...]) + _dot(h1, w11_ref[...])) * dinv


def _tc_final_body(c0_ref, c1_ref, dinv_ref, b_ref,
                   wf1a_ref, wf1b_ref, bf1_ref, wf2_ref, bf2_ref, o_ref):
    dinv = dinv_ref[...]
    b = b_ref[...]
    h0 = jnp.maximum(c0_ref[...] * dinv + b[0:1, :], 0.0)
    h1 = jnp.maximum(c1_ref[...] * dinv + b[1:2, :], 0.0)
    f = _dot(h0, wf1a_ref[...]) + _dot(h1, wf1b_ref[...]) + bf1_ref[...]
    f = jnp.maximum(f, 0.0)
    o_ref[...] = _dot(f, wf2_ref[...]) + bf2_ref[...]


def _tc_prologue(d0, d1, x0, x1, t_w1):
    return pl.pallas_call(
        _tc_prologue_body,
        grid=(GRID_P,),
        in_specs=[_prow_spec(), _prow_spec(), _prow_spec(), _prow_spec(),
                  _full_spec((4, 128))],
        out_specs=(_prow_spec(), _prow_spec(), _prow_spec()),
        out_shape=(
            jax.ShapeDtypeStruct((N_P4, 128), jnp.float32),
            jax.ShapeDtypeStruct((N_P4, 128), jnp.float32),
            jax.ShapeDtypeStruct((N_P4, 128), jnp.float32),
        ),
    )(d0, d1, x0, x1, t_w1)


def _tc_mid(a0, a1, dinv, b1t, w00, w01, w10, w11):
    return pl.pallas_call(
        _tc_mid_body,
        grid=(GRID_P,),
        in_specs=[_prow_spec(), _prow_spec(), _prow_spec(),
                  _full_spec((2, 128)), _full_spec((128, 128)),
                  _full_spec((128, 128)), _full_spec((128, 128)),
                  _full_spec((128, 128))],
        out_specs=(_prow_spec(), _prow_spec()),
        out_shape=(
            jax.ShapeDtypeStruct((N_P4, 128), jnp.float32),
            jax.ShapeDtypeStruct((N_P4, 128), jnp.float32),
        ),
    )(a0, a1, dinv, b1t, w00, w01, w10, w11)


def _tc_final(c0, c1, dinv, b2t, wf1a, wf1b, bf1t, wf2bd, bf2t):
    return pl.pallas_call(
        _tc_final_body,
        grid=(GRID_P,),
        in_specs=[_prow_spec(), _prow_spec(), _prow_spec(),
                  _full_spec((2, 128)),
                  _full_spec((128, 4 * FFN_D)), _full_spec((128, 4 * FFN_D)),
                  _full_spec((1, 4 * FFN_D)),
                  _full_spec((4 * FFN_D, 128)), _full_spec((1, 128))],
        out_specs=_prow_spec(),
        out_shape=jax.ShapeDtypeStruct((N_P4, 128), jnp.float32),
    )(c0, c1, dinv, b2t, wf1a, wf1b, bf1t, wf2bd, bf2t)


def kernel(x, edge_index, W1, b1, W2, b2, Wf1, bf1, Wf2, bf2):
    f32 = jnp.float32
    x = x.astype(f32)
    src = edge_index[0].astype(jnp.int32)
    dst = edge_index[1].astype(jnp.int32)

    pad = jnp.int32(N) + (jnp.arange(E_PAD - E, dtype=jnp.int32) % (N_PAD - N))
    src_p = jnp.concatenate([src, pad])
    dst_p = jnp.concatenate([dst, pad])
    src16 = src_p.reshape(16, ROWS16, CHUNK)
    dst16 = dst_p.reshape(16, ROWS16, CHUNK)
    dst32 = dst_p.reshape(32, ROWS32, CHUNK)

    ones32 = jnp.ones((CHUNK, HALF), f32)
    zeros32 = jnp.zeros((STG, HALF), f32)

    xcol = jnp.concatenate([x, jnp.zeros((N_PAD - N, 2), f32)])
    xp0 = jnp.broadcast_to(xcol[:, 0:1], (N_PAD, HALF)).reshape(N_P4, 128)
    xp1 = jnp.broadcast_to(xcol[:, 1:2], (N_PAD, HALF)).reshape(N_P4, 128)

    W1f, W2f = W1.astype(f32), W2.astype(f32)
    Wf1f, Wf2f = Wf1.astype(f32), Wf2.astype(f32)
    b1f, b2f = b1.astype(f32), b2.astype(f32)
    bf1f, bf2f = bf1.astype(f32), bf2.astype(f32)
    eye4 = jnp.eye(4, dtype=f32)

    def bd4(a):
        return jnp.kron(eye4, a)

    t_w1 = jnp.stack([
        jnp.tile(W1f[0, :HALF], 4), jnp.tile(W1f[1, :HALF], 4),
        jnp.tile(W1f[0, HALF:], 4), jnp.tile(W1f[1, HALF:], 4)])
    w00 = bd4(W2f[:HALF, :HALF])
    w01 = bd4(W2f[:HALF, HALF:])
    w10 = bd4(W2f[HALF:, :HALF])
    w11 = bd4(W2f[HALF:, HALF:])
    b1t = jnp.stack([jnp.tile(b1f[:HALF], 4), jnp.tile(b1f[HALF:], 4)])
    b2t = jnp.stack([jnp.tile(b2f[:HALF], 4), jnp.tile(b2f[HALF:], 4)])
    wf1a = bd4(Wf1f[:HALF, :])
    wf1b = bd4(Wf1f[HALF:, :])
    bf1t = jnp.tile(bf1f, 4).reshape(1, 4 * FFN_D)
    wf2bd = bd4(Wf2f)
    bf2t = jnp.tile(bf2f, 4).reshape(1, 128)

    deg0, deg1 = _sc_degree(dst32, ones32, zeros32)
    dinv, y0p, y1p = _tc_prologue(deg0.reshape(N_P4, 128),
                                  deg1.reshape(N_P4, 128), xp0, xp1, t_w1)

    a0, a1 = _sc_aggregate(y0p.reshape(N_PAD, HALF), y1p.reshape(N_PAD, HALF),
                           src16, dst16)
    z0p, z1p = _tc_mid(a0.reshape(N_P4, 128), a1.reshape(N_P4, 128),
                       dinv, b1t, w00, w01, w10, w11)
    c0, c1 = _sc_aggregate(z0p.reshape(N_PAD, HALF), z1p.reshape(N_PAD, HALF),
                           src16, dst16)
    outp = _tc_final(c0.reshape(N_P4, 128), c1.reshape(N_P4, 128),
                     dinv, b2t, wf1a, wf1b, bf1t, wf2bd, bf2t)
    return outp.reshape(N_PAD, OUT_D)[:N].astype(jnp.float64)
